# Initial kernel scaffold; baseline (speedup 1.0000x reference)
import jax
import jax.numpy as jnp
from jax.experimental import pallas as pl


def kernel(feats, edge_index, key_table, val_table, W1, b1, W2, b2, Wc):
    raise NotImplementedError("write your pallas kernel here")


# speedup vs baseline: 3.7319x; 3.7319x over previous
import functools

import jax
import jax.numpy as jnp
from jax import lax
from jax.experimental import pallas as pl
from jax.experimental.pallas import tpu as pltpu
from jax.experimental.pallas import tpu_sc as plsc

N = 10000
NP = 10240
E = 320000
EPAD = 323584
H = 128
VOCAB = 1001

ROWS_PER_TILE_A = NP // 32
ROWS_PER_TILE_B = NP // 16
EDGES_PER_TILE = EPAD // 32
EDGE_CHUNKS = EDGES_PER_TILE // 128

_mesh = plsc.VectorSubcoreMesh(core_axis_name="c", subcore_axis_name="s")


@functools.partial(
    pl.kernel,
    mesh=_mesh,
    out_type=jax.ShapeDtypeStruct((NP, H), jnp.float32),
    scratch_types=[
        pltpu.VMEM((ROWS_PER_TILE_A,), jnp.int32),
        pltpu.VMEM((ROWS_PER_TILE_A,), jnp.int32),
        pltpu.VMEM((64, H), jnp.float32),
        pltpu.VMEM((64, H), jnp.float32),
        pltpu.SemaphoreType.DMA,
    ],
)
def _sc_embed(f0_hbm, f1_hbm, kt_hbm, vt_hbm, h_hbm, fi0, fi1, kbuf, vbuf, sem):
    c = lax.axis_index("c")
    s = lax.axis_index("s")
    wid = s * 2 + c
    n0 = wid * ROWS_PER_TILE_A
    pltpu.sync_copy(f0_hbm.at[pl.ds(n0, ROWS_PER_TILE_A)], fi0)
    pltpu.sync_copy(f1_hbm.at[pl.ds(n0, ROWS_PER_TILE_A)], fi1)
    for i in range(ROWS_PER_TILE_A // 64):
        pltpu.async_copy(kt_hbm.at[fi0.at[pl.ds(64 * i, 64)]], kbuf, sem).wait()
        pltpu.async_copy(vt_hbm.at[fi1.at[pl.ds(64 * i, 64)]], vbuf, sem).wait()

        def relu_body(r, carry):
            for j in range(H // 16):
                kbuf[r, pl.ds(16 * j, 16)] = jnp.maximum(
                    kbuf[r, pl.ds(16 * j, 16)] + vbuf[r, pl.ds(16 * j, 16)],
                    0.0)
            return carry
        lax.fori_loop(0, 64, relu_body, 0)
        pltpu.sync_copy(kbuf, h_hbm.at[pl.ds(n0 + 64 * i, 64)])


@functools.partial(
    pl.kernel,
    mesh=_mesh,
    out_type=jax.ShapeDtypeStruct((2, NP, H), jnp.float32),
    scratch_types=[
        pltpu.VMEM((128,), jnp.int32),
        pltpu.VMEM((128,), jnp.int32),
        pltpu.VMEM((128, H), jnp.float32),
        pltpu.VMEM_SHARED((NP, H), jnp.float32),
        pltpu.SemaphoreType.DMA,
    ],
)
def _sc_edges(h_hbm, src_hbm, dst_hbm, out_hbm, esrc, edst, erows, acc_sh, sem):
    c = lax.axis_index("c")
    s = lax.axis_index("s")
    r0 = s * ROWS_PER_TILE_B

    @pl.when(c == 0)
    def _():
        pltpu.sync_copy(h_hbm.at[pl.ds(r0, ROWS_PER_TILE_B)],
                        acc_sh.at[pl.ds(r0, ROWS_PER_TILE_B)])

    @pl.when(c == 1)
    def _():
        def zero_body(r, carry):
            for j in range(H // 16):
                erows[r, pl.ds(16 * j, 16)] = jnp.zeros((16,), jnp.float32)
            return carry
        lax.fori_loop(0, 128, zero_body, 0)
        for i in range(ROWS_PER_TILE_B // 128):
            pltpu.sync_copy(erows, acc_sh.at[pl.ds(r0 + 128 * i, 128)])

    plsc.subcore_barrier()

    e0 = (c * 16 + s) * EDGES_PER_TILE

    def edge_body(j, carry):
        base = e0 + j * 128
        pltpu.sync_copy(src_hbm.at[pl.ds(base, 128)], esrc)
        pltpu.sync_copy(dst_hbm.at[pl.ds(base, 128)], edst)
        pltpu.async_copy(h_hbm.at[esrc], erows, sem).wait()
        pltpu.sync_copy(erows, acc_sh.at[edst], add=True)
        return carry
    lax.fori_loop(0, EDGE_CHUNKS, edge_body, 0)

    plsc.subcore_barrier()

    for i in range(ROWS_PER_TILE_B // 128):
        pltpu.sync_copy(acc_sh.at[pl.ds(r0 + 128 * i, 128)], erows)
        pltpu.sync_copy(erows, out_hbm.at[c].at[pl.ds(r0 + 128 * i, 128)])


BLK = 1024


def _mlp_body(ha_ref, hb_ref, w1_ref, b1_ref, w2_ref, b2_ref, wc_ref, o_ref):
    h = ha_ref[0] + hb_ref[0]
    z = jnp.dot(h, w1_ref[...], preferred_element_type=jnp.float32)
    z = jnp.maximum(z + b1_ref[...], 0.0)
    z = jnp.dot(z, w2_ref[...], preferred_element_type=jnp.float32) + b2_ref[...]
    o_ref[...] = jnp.dot(z, wc_ref[...], preferred_element_type=jnp.float32)


def _mlp(hs, W1, b1, W2, b2, Wc):
    return pl.pallas_call(
        _mlp_body,
        grid=(NP // BLK,),
        in_specs=[
            pl.BlockSpec((1, BLK, H), lambda i: (0, i, 0)),
            pl.BlockSpec((1, BLK, H), lambda i: (1, i, 0)),
            pl.BlockSpec((H, H), lambda i: (0, 0)),
            pl.BlockSpec((1, H), lambda i: (0, 0)),
            pl.BlockSpec((H, H), lambda i: (0, 0)),
            pl.BlockSpec((1, H), lambda i: (0, 0)),
            pl.BlockSpec((H, H), lambda i: (0, 0)),
        ],
        out_specs=pl.BlockSpec((BLK, H), lambda i: (i, 0)),
        out_shape=jax.ShapeDtypeStruct((NP, H), jnp.float32),
    )(hs, hs, W1, b1.reshape(1, H), W2, b2.reshape(1, H), Wc)


def kernel(feats, edge_index, key_table, val_table, W1, b1, W2, b2, Wc):
    f0 = jnp.pad(feats[:, 0], (0, NP - N))
    f1 = jnp.pad(feats[:, 1], (0, NP - N))
    srcp = jnp.full((EPAD,), NP - 1, jnp.int32).at[:E].set(edge_index[0])
    dstp = jnp.full((EPAD,), NP - 1, jnp.int32).at[:E].set(edge_index[1])
    h = _sc_embed(f0, f1, key_table, val_table)
    hs = _sc_edges(h, srcp, dstp)
    out = _mlp(hs, W1, b1, W2, b2, Wc)
    return out[:N]
